# baseline (speedup 1.0000x reference)
import functools
import jax
import jax.numpy as jnp
from jax import lax
from jax.experimental import pallas as pl
from jax.experimental.pallas import tpu as pltpu
from jax.experimental.pallas import tpu_sc as plsc

NC = 2
NS = 16
L = 16
CHUNK = 128
K_HOPS = 3

_MESH = dict(core_axis_name="c", subcore_axis_name="s",
             num_cores=NC, num_subcores=NS)


def _full16(v):
    return jnp.full((L,), v, dtype=jnp.int32)


def _eas_body(ea_ref, out_ref):
    sel = (lax.broadcasted_iota(jnp.int32, (128, 32), 0) // 4
           == lax.broadcasted_iota(jnp.int32, (128, 32), 1)).astype(jnp.float32)
    out_ref[...] = jnp.dot(ea_ref[...], sel, preferred_element_type=jnp.float32)


def _dis_body(dp_ref, dis_ref):
    rows = dp_ref.shape[0] // 2
    deg = dp_ref[:rows, :] + dp_ref[rows:, :]
    safe = jnp.where(deg > 0, deg, 1.0)
    dis_ref[...] = jnp.where(deg > 0, lax.rsqrt(safe), 0.0)


def _deg_kernel(n_pad, npt, nch_sc, col_hbm, val_hbm, part_hbm,
                col_v, val_v, zero_v, acc_sh):
    cid = lax.axis_index("c")
    sid = lax.axis_index("s")
    for i in range(npt // L):
        zero_v[pl.ds(i * L, L)] = jnp.zeros((L,), jnp.float32)
    pltpu.sync_copy(zero_v, acc_sh.at[pl.ds(sid * npt, npt)])
    plsc.subcore_barrier()

    base = cid * nch_sc
    pltpu.sync_copy(col_hbm.at[sid].at[pl.ds(base, nch_sc)], col_v)
    pltpu.sync_copy(val_hbm.at[sid].at[pl.ds(base, nch_sc)], val_v)

    def body(j, _):
        pltpu.sync_copy(val_v.at[j], acc_sh.at[col_v.at[j]], add=True)
        return ()
    lax.fori_loop(0, nch_sc, body, (), unroll=False)

    plsc.subcore_barrier()
    pltpu.sync_copy(acc_sh.at[pl.ds(sid * npt, npt)],
                    part_hbm.at[cid, pl.ds(sid * npt, npt)])


def _w_kernel(nch_sc, row_hbm, col_hbm, eas_hbm, dis_hbm, w_hbm,
              row_v, col_v, eas_v, dis_v, w_v):
    cid = lax.axis_index("c")
    sid = lax.axis_index("s")
    base = cid * nch_sc
    pltpu.sync_copy(dis_hbm, dis_v)
    pltpu.sync_copy(row_hbm.at[sid].at[pl.ds(base, nch_sc)], row_v)
    pltpu.sync_copy(col_hbm.at[sid].at[pl.ds(base, nch_sc)], col_v)
    pltpu.sync_copy(eas_hbm.at[sid].at[pl.ds(base, nch_sc)], eas_v)

    def body(j, _):
        for g in range(CHUNK // L):
            sl = pl.ds(g * L, L)
            r16 = row_v[j, sl]
            c16 = col_v[j, sl]
            dr = plsc.load_gather(dis_v, [r16])
            dc = plsc.load_gather(dis_v, [c16])
            w_v[j, sl] = dr * eas_v[j, sl] * dc
        return ()
    lax.fori_loop(0, nch_sc, body, (), unroll=False)
    pltpu.sync_copy(w_v, w_hbm.at[sid].at[pl.ds(base, nch_sc)])


def kernel(x, edge_index, edge_attr):
    n, d = x.shape
    e = edge_index.shape[1]
    dh = d // NC
    row = edge_index[0]
    col = edge_index[1]

    ept = ((e + NS * CHUNK - 1) // (NS * CHUNK)) * CHUNK
    nchunks = ept // CHUNK
    nchunks = ((nchunks + 15) // 16) * 16
    ept = nchunks * CHUNK
    e_pad = ept * NS
    nch_sc = nchunks // 2
    npt = ((n + NS * L - 1) // (NS * L)) * L
    n_pad = npt * NS

    row_p = jnp.pad(row, (0, e_pad - e)).reshape(NS, nchunks, CHUNK)
    col_p = jnp.pad(col, (0, e_pad - e)).reshape(NS, nchunks, CHUNK)
    ea_p = jnp.pad(edge_attr, ((0, e_pad - e), (0, 0)))
    x_split = jnp.pad(
        jnp.stack([x[:, i * dh:(i + 1) * dh] for i in range(NC)], axis=0),
        ((0, 0), (0, n_pad - n), (0, 0)))

    eas = pl.pallas_call(
        _eas_body,
        out_shape=jax.ShapeDtypeStruct((e_pad // 32, 32), jnp.float32),
    )(ea_p.reshape(e_pad // 32, 128))
    eas_w = eas.reshape(NS, nchunks, CHUNK)

    deg_part = pl.kernel(
        functools.partial(_deg_kernel, n_pad, npt, nch_sc),
        out_type=jax.ShapeDtypeStruct((NC, n_pad), jnp.float32),
        mesh=plsc.VectorSubcoreMesh(**_MESH),
        compiler_params=pltpu.CompilerParams(needs_layout_passes=False, use_tc_tiling_on_sc=False),
        scratch_types=[
            pltpu.VMEM((nch_sc, CHUNK), jnp.int32),
            pltpu.VMEM((nch_sc, CHUNK), jnp.float32),
            pltpu.VMEM((npt,), jnp.float32),
            pltpu.VMEM_SHARED((n_pad,), jnp.float32),
        ],
    )(col_p, eas_w)

    dis = pl.pallas_call(
        _dis_body,
        out_shape=jax.ShapeDtypeStruct((n_pad // 128, 128), jnp.float32),
    )(deg_part.reshape(2 * (n_pad // 128), 128)).reshape(n_pad)

    w = pl.kernel(
        functools.partial(_w_kernel, nch_sc),
        out_type=jax.ShapeDtypeStruct((NS, nchunks, CHUNK), jnp.float32),
        mesh=plsc.VectorSubcoreMesh(**_MESH),
        compiler_params=pltpu.CompilerParams(needs_layout_passes=False, use_tc_tiling_on_sc=False),
        scratch_types=[
            pltpu.VMEM((nch_sc, CHUNK), jnp.int32),
            pltpu.VMEM((nch_sc, CHUNK), jnp.int32),
            pltpu.VMEM((nch_sc, CHUNK), jnp.float32),
            pltpu.VMEM((n_pad,), jnp.float32),
            pltpu.VMEM((nch_sc, CHUNK), jnp.float32),
        ],
    )(row_p, col_p, eas_w, dis)

    def _hop_body(x_hbm, row_hbm, col_hbm, w_hbm, part_hbm,
                  row_v, col_v, w_v, buf0, buf1, gs0, gs1, ss0, ss1, acc_sh):
        cid = lax.axis_index("c")
        sid = lax.axis_index("s")

        def zfill(i, _):
            for g in range(dh // L):
                buf0[i, pl.ds(g * L, L)] = jnp.zeros((L,), jnp.float32)
            return ()
        lax.fori_loop(0, CHUNK, zfill, (), unroll=False)

        def zbody(i, _):
            pltpu.sync_copy(
                buf0, acc_sh.at[pl.ds(sid * npt + i * CHUNK, CHUNK)])
            return ()
        lax.fori_loop(0, npt // CHUNK, zbody, (), unroll=False)
        plsc.subcore_barrier()

        pltpu.sync_copy(row_hbm.at[sid], row_v)
        pltpu.sync_copy(col_hbm.at[sid], col_v)
        pltpu.sync_copy(w_hbm.at[sid], w_v)

        xc = x_hbm.at[cid]
        bufs = (buf0, buf1)
        gsems = (gs0, gs1)
        ssems = (ss0, ss1)

        def scale(j, buf):
            def grp(g, _):
                for i2 in range(L):
                    r = g * L + i2
                    wb = plsc.load_gather(w_v, [_full16(j), _full16(r)])
                    for dd in range(dh // L):
                        sl = pl.ds(dd * L, L)
                        buf[r, sl] = buf[r, sl] * wb
                return ()
            lax.fori_loop(0, CHUNK // L, grp, (), unroll=False)

        pltpu.async_copy(xc.at[row_v.at[0]], buf0, gs0)

        def pair(jj, _):
            for b in range(2):
                j = jj * 2 + b
                buf, gs, ss = bufs[b], gsems[b], ssems[b]
                nb, ngs = bufs[1 - b], gsems[1 - b]

                if b == 0:
                    @pl.when(jj >= 1)
                    def _():
                        pltpu.make_async_copy(
                            nb, acc_sh.at[col_v.at[j - 1]], ssems[1 - b]
                        ).wait()

                    pltpu.async_copy(xc.at[row_v.at[j + 1]], nb, ngs)
                else:
                    pltpu.make_async_copy(
                        nb, acc_sh.at[col_v.at[j - 1]], ssems[1 - b]).wait()

                    @pl.when(jj < nchunks // 2 - 1)
                    def _():
                        pltpu.async_copy(xc.at[row_v.at[j + 1]], nb, ngs)

                pltpu.make_async_copy(xc.at[row_v.at[j]], buf, gs).wait()
                scale(j, buf)
                pltpu.async_copy(buf, acc_sh.at[col_v.at[j]], ss, add=True)
            return ()
        lax.fori_loop(0, nchunks // 2, pair, (), unroll=False)
        pltpu.make_async_copy(
            buf1, acc_sh.at[col_v.at[nchunks - 1]], ss1).wait()

        plsc.subcore_barrier()
        pltpu.sync_copy(acc_sh.at[pl.ds(sid * npt, npt)],
                        part_hbm.at[cid, pl.ds(sid * npt, npt)])

    hop = pl.kernel(
        _hop_body,
        out_type=jax.ShapeDtypeStruct((NC, n_pad, dh), jnp.float32),
        mesh=plsc.VectorSubcoreMesh(**_MESH),
        compiler_params=pltpu.CompilerParams(needs_layout_passes=False, use_tc_tiling_on_sc=False),
        scratch_types=[
            pltpu.VMEM((nchunks, CHUNK), jnp.int32),
            pltpu.VMEM((nchunks, CHUNK), jnp.int32),
            pltpu.VMEM((nchunks, CHUNK), jnp.float32),
            pltpu.VMEM((CHUNK, dh), jnp.float32),
            pltpu.VMEM((CHUNK, dh), jnp.float32),
            pltpu.SemaphoreType.DMA,
            pltpu.SemaphoreType.DMA,
            pltpu.SemaphoreType.DMA,
            pltpu.SemaphoreType.DMA,
            pltpu.VMEM_SHARED((n_pad, dh), jnp.float32),
        ],
    )

    xs = [x]
    cur = x_split
    for _ in range(K_HOPS):
        cur = hop(cur, row_p, col_p, w)
        xs.append(jnp.concatenate([cur[i, :n] for i in range(NC)], axis=1))
    return jnp.stack(xs, axis=0)


# speedup vs baseline: 1.0079x; 1.0079x over previous
import functools
import jax
import jax.numpy as jnp
from jax import lax
from jax.experimental import pallas as pl
from jax.experimental.pallas import tpu as pltpu
from jax.experimental.pallas import tpu_sc as plsc

NC = 2
NS = 16
L = 16
CHUNK = 128
K_HOPS = 3

_MESH = dict(core_axis_name="c", subcore_axis_name="s",
             num_cores=NC, num_subcores=NS)


def _full16(v):
    return jnp.full((L,), v, dtype=jnp.int32)


def _eas_body(ea_ref, out_ref):
    sel = (lax.broadcasted_iota(jnp.int32, (128, 32), 0) // 4
           == lax.broadcasted_iota(jnp.int32, (128, 32), 1)).astype(jnp.float32)
    out_ref[...] = jnp.dot(ea_ref[...], sel, preferred_element_type=jnp.float32)


def _dis_body(dp_ref, dis_ref):
    rows = dp_ref.shape[0] // 2
    deg = dp_ref[:rows, :] + dp_ref[rows:, :]
    safe = jnp.where(deg > 0, deg, 1.0)
    dis_ref[...] = jnp.where(deg > 0, lax.rsqrt(safe), 0.0)


def _deg_kernel(n_pad, npt, nch_sc, col_hbm, val_hbm, part_hbm,
                col_v, val_v, zero_v, acc_sh):
    cid = lax.axis_index("c")
    sid = lax.axis_index("s")
    for i in range(npt // L):
        zero_v[pl.ds(i * L, L)] = jnp.zeros((L,), jnp.float32)
    pltpu.sync_copy(zero_v, acc_sh.at[pl.ds(sid * npt, npt)])
    plsc.subcore_barrier()

    base = cid * nch_sc
    pltpu.sync_copy(col_hbm.at[sid].at[pl.ds(base, nch_sc)], col_v)
    pltpu.sync_copy(val_hbm.at[sid].at[pl.ds(base, nch_sc)], val_v)

    def body(j, _):
        pltpu.sync_copy(val_v.at[j], acc_sh.at[col_v.at[j]], add=True)
        return ()
    lax.fori_loop(0, nch_sc, body, (), unroll=False)

    plsc.subcore_barrier()
    pltpu.sync_copy(acc_sh.at[pl.ds(sid * npt, npt)],
                    part_hbm.at[cid, pl.ds(sid * npt, npt)])


def _w_kernel(nch_sc, row_hbm, col_hbm, eas_hbm, dis_hbm, w_hbm,
              row_v, col_v, eas_v, dis_v, w_v):
    cid = lax.axis_index("c")
    sid = lax.axis_index("s")
    base = cid * nch_sc
    pltpu.sync_copy(dis_hbm, dis_v)
    pltpu.sync_copy(row_hbm.at[sid].at[pl.ds(base, nch_sc)], row_v)
    pltpu.sync_copy(col_hbm.at[sid].at[pl.ds(base, nch_sc)], col_v)
    pltpu.sync_copy(eas_hbm.at[sid].at[pl.ds(base, nch_sc)], eas_v)

    def body(j, _):
        for g in range(CHUNK // L):
            sl = pl.ds(g * L, L)
            r16 = row_v[j, sl]
            c16 = col_v[j, sl]
            dr = plsc.load_gather(dis_v, [r16])
            dc = plsc.load_gather(dis_v, [c16])
            w_v[j, sl] = dr * eas_v[j, sl] * dc
        return ()
    lax.fori_loop(0, nch_sc, body, (), unroll=False)
    pltpu.sync_copy(w_v, w_hbm.at[sid].at[pl.ds(base, nch_sc)])


def kernel(x, edge_index, edge_attr):
    n, d = x.shape
    e = edge_index.shape[1]
    dh = d // NC
    row = edge_index[0]
    col = edge_index[1]

    ept = ((e + NS * CHUNK - 1) // (NS * CHUNK)) * CHUNK
    nchunks = ept // CHUNK
    nchunks = ((nchunks + 15) // 16) * 16
    ept = nchunks * CHUNK
    e_pad = ept * NS
    nch_sc = nchunks // 2
    npt = ((n + NS * L - 1) // (NS * L)) * L
    n_pad = npt * NS

    row_p = jnp.pad(row, (0, e_pad - e)).reshape(NS, nchunks, CHUNK)
    col_p = jnp.pad(col, (0, e_pad - e)).reshape(NS, nchunks, CHUNK)
    ea_p = jnp.pad(edge_attr, ((0, e_pad - e), (0, 0)))
    x_split = jnp.pad(
        jnp.stack([x[:, i * dh:(i + 1) * dh] for i in range(NC)], axis=0),
        ((0, 0), (0, n_pad - n), (0, 0)))

    eas = pl.pallas_call(
        _eas_body,
        out_shape=jax.ShapeDtypeStruct((e_pad // 32, 32), jnp.float32),
    )(ea_p.reshape(e_pad // 32, 128))
    eas_w = eas.reshape(NS, nchunks, CHUNK)

    deg_part = pl.kernel(
        functools.partial(_deg_kernel, n_pad, npt, nch_sc),
        out_type=jax.ShapeDtypeStruct((NC, n_pad), jnp.float32),
        mesh=plsc.VectorSubcoreMesh(**_MESH),
        compiler_params=pltpu.CompilerParams(needs_layout_passes=False, use_tc_tiling_on_sc=False),
        scratch_types=[
            pltpu.VMEM((nch_sc, CHUNK), jnp.int32),
            pltpu.VMEM((nch_sc, CHUNK), jnp.float32),
            pltpu.VMEM((npt,), jnp.float32),
            pltpu.VMEM_SHARED((n_pad,), jnp.float32),
        ],
    )(col_p, eas_w)

    dis = pl.pallas_call(
        _dis_body,
        out_shape=jax.ShapeDtypeStruct((n_pad // 128, 128), jnp.float32),
    )(deg_part.reshape(2 * (n_pad // 128), 128)).reshape(n_pad)

    w = pl.kernel(
        functools.partial(_w_kernel, nch_sc),
        out_type=jax.ShapeDtypeStruct((NS, nchunks, CHUNK), jnp.float32),
        mesh=plsc.VectorSubcoreMesh(**_MESH),
        compiler_params=pltpu.CompilerParams(needs_layout_passes=False, use_tc_tiling_on_sc=False),
        scratch_types=[
            pltpu.VMEM((nch_sc, CHUNK), jnp.int32),
            pltpu.VMEM((nch_sc, CHUNK), jnp.int32),
            pltpu.VMEM((nch_sc, CHUNK), jnp.float32),
            pltpu.VMEM((n_pad,), jnp.float32),
            pltpu.VMEM((nch_sc, CHUNK), jnp.float32),
        ],
    )(row_p, col_p, eas_w, dis)

    NBUF = 4
    NSEG = 2
    nch_seg = nchunks // NSEG

    def _hop_body(x_hbm, row_hbm, col_hbm, w_hbm, part_hbm,
                  row_v, col_v, w_v, bufs, gsems, ssems, acc_sh):
        cid = lax.axis_index("c")
        sid = lax.axis_index("s")

        def zfill(i, _):
            for g in range(dh // L):
                bufs[0][i, pl.ds(g * L, L)] = jnp.zeros((L,), jnp.float32)
            return ()
        lax.fori_loop(0, CHUNK, zfill, (), unroll=False)

        def zbody(i, _):
            pltpu.sync_copy(
                bufs[0], acc_sh.at[pl.ds(sid * npt + i * CHUNK, CHUNK)])
            return ()
        lax.fori_loop(0, npt // CHUNK, zbody, (), unroll=False)
        plsc.subcore_barrier()

        xc = x_hbm.at[cid]

        def scale(j, buf):
            def grp(g, _):
                for i2 in range(L):
                    r = g * L + i2
                    wb = plsc.load_gather(w_v, [_full16(j), _full16(r)])
                    for dd in range(dh // L):
                        sl = pl.ds(dd * L, L)
                        buf[r, sl] = buf[r, sl] * wb
                return ()
            lax.fori_loop(0, CHUNK // L, grp, (), unroll=False)

        for seg in range(NSEG):
            cbase = seg * nch_seg
            pltpu.sync_copy(row_hbm.at[sid].at[pl.ds(cbase, nch_seg)], row_v)
            pltpu.sync_copy(col_hbm.at[sid].at[pl.ds(cbase, nch_seg)], col_v)
            pltpu.sync_copy(w_hbm.at[sid].at[pl.ds(cbase, nch_seg)], w_v)

            for b in range(NBUF - 1):
                pltpu.async_copy(xc.at[row_v.at[b]], bufs[b], gsems[b])

            def quad(jj, _):
                for b in range(NBUF):
                    j = jj * NBUF + b
                    buf, gs, ss = bufs[b], gsems[b], ssems[b]
                    pb = (b + NBUF - 1) % NBUF

                    def wait_prev():
                        pltpu.make_async_copy(
                            bufs[pb], acc_sh.at[col_v.at[j - 1]],
                            ssems[pb]).wait()

                    def prefetch():
                        pltpu.async_copy(
                            xc.at[row_v.at[j + NBUF - 1]], bufs[pb],
                            gsems[pb])

                    if b == 0:
                        @pl.when(jj >= 1)
                        def _():
                            wait_prev()
                        prefetch()
                    else:
                        wait_prev()

                        @pl.when(jj < nch_seg // NBUF - 1)
                        def _():
                            prefetch()

                    pltpu.make_async_copy(xc.at[row_v.at[j]], buf, gs).wait()
                    scale(j, buf)
                    pltpu.async_copy(buf, acc_sh.at[col_v.at[j]], ss,
                                     add=True)
                return ()
            lax.fori_loop(0, nch_seg // NBUF, quad, (), unroll=False)
            lb = (nch_seg - 1) % NBUF
            pltpu.make_async_copy(
                bufs[lb], acc_sh.at[col_v.at[nch_seg - 1]],
                ssems[lb]).wait()

        plsc.subcore_barrier()
        pltpu.sync_copy(acc_sh.at[pl.ds(sid * npt, npt)],
                        part_hbm.at[cid, pl.ds(sid * npt, npt)])

    hop = pl.kernel(
        _hop_body,
        out_type=jax.ShapeDtypeStruct((NC, n_pad, dh), jnp.float32),
        mesh=plsc.VectorSubcoreMesh(**_MESH),
        compiler_params=pltpu.CompilerParams(needs_layout_passes=False, use_tc_tiling_on_sc=False),
        scratch_types=[
            pltpu.VMEM((nch_seg, CHUNK), jnp.int32),
            pltpu.VMEM((nch_seg, CHUNK), jnp.int32),
            pltpu.VMEM((nch_seg, CHUNK), jnp.float32),
            [pltpu.VMEM((CHUNK, dh), jnp.float32) for _ in range(NBUF)],
            [pltpu.SemaphoreType.DMA for _ in range(NBUF)],
            [pltpu.SemaphoreType.DMA for _ in range(NBUF)],
            pltpu.VMEM_SHARED((n_pad, dh), jnp.float32),
        ],
    )

    xs = [x]
    cur = x_split
    for _ in range(K_HOPS):
        cur = hop(cur, row_p, col_p, w)
        xs.append(jnp.concatenate([cur[i, :n] for i in range(NC)], axis=1))
    return jnp.stack(xs, axis=0)


# speedup vs baseline: 1.4189x; 1.4078x over previous
import functools
import jax
import jax.numpy as jnp
from jax import lax
from jax.experimental import pallas as pl
from jax.experimental.pallas import tpu as pltpu
from jax.experimental.pallas import tpu_sc as plsc

NC = 2
NS = 16
L = 16
CHUNK = 128
K_HOPS = 3

_MESH = dict(core_axis_name="c", subcore_axis_name="s",
             num_cores=NC, num_subcores=NS)


def _full16(v):
    return jnp.full((L,), v, dtype=jnp.int32)


def _eas_body(ea_ref, out_ref):
    sel = (lax.broadcasted_iota(jnp.int32, (128, 32), 0) // 4
           == lax.broadcasted_iota(jnp.int32, (128, 32), 1)).astype(jnp.float32)
    out_ref[...] = jnp.dot(ea_ref[...], sel, preferred_element_type=jnp.float32)


def _dis_body(dp_ref, dis_ref):
    rows = dp_ref.shape[0] // 2
    deg = dp_ref[:rows, :] + dp_ref[rows:, :]
    safe = jnp.where(deg > 0, deg, 1.0)
    dis_ref[...] = jnp.where(deg > 0, lax.rsqrt(safe), 0.0)


def _deg_kernel(n_pad, npt, nch_sc, col_hbm, val_hbm, part_hbm,
                col_v, val_v, zero_v, acc_sh):
    cid = lax.axis_index("c")
    sid = lax.axis_index("s")
    for i in range(npt // L):
        zero_v[pl.ds(i * L, L)] = jnp.zeros((L,), jnp.float32)
    pltpu.sync_copy(zero_v, acc_sh.at[pl.ds(sid * npt, npt)])
    plsc.subcore_barrier()

    base = cid * nch_sc
    pltpu.sync_copy(col_hbm.at[sid].at[pl.ds(base, nch_sc)], col_v)
    pltpu.sync_copy(val_hbm.at[sid].at[pl.ds(base, nch_sc)], val_v)

    def body(j, _):
        pltpu.sync_copy(val_v.at[j], acc_sh.at[col_v.at[j]], add=True)
        return ()
    lax.fori_loop(0, nch_sc, body, (), unroll=False)

    plsc.subcore_barrier()
    pltpu.sync_copy(acc_sh.at[pl.ds(sid * npt, npt)],
                    part_hbm.at[cid, pl.ds(sid * npt, npt)])


def _w_kernel(nch_sc, row_hbm, col_hbm, eas_hbm, dis_hbm, w_hbm,
              row_v, col_v, eas_v, dis_v, w_v):
    cid = lax.axis_index("c")
    sid = lax.axis_index("s")
    base = cid * nch_sc
    pltpu.sync_copy(dis_hbm, dis_v)
    pltpu.sync_copy(row_hbm.at[sid].at[pl.ds(base, nch_sc)], row_v)
    pltpu.sync_copy(col_hbm.at[sid].at[pl.ds(base, nch_sc)], col_v)
    pltpu.sync_copy(eas_hbm.at[sid].at[pl.ds(base, nch_sc)], eas_v)

    def body(j, _):
        for g in range(CHUNK // L):
            sl = pl.ds(g * L, L)
            r16 = row_v[j, sl]
            c16 = col_v[j, sl]
            dr = plsc.load_gather(dis_v, [r16])
            dc = plsc.load_gather(dis_v, [c16])
            w_v[j, sl] = dr * eas_v[j, sl] * dc
        return ()
    lax.fori_loop(0, nch_sc, body, (), unroll=False)
    pltpu.sync_copy(w_v, w_hbm.at[sid].at[pl.ds(base, nch_sc)])


def kernel(x, edge_index, edge_attr):
    n, d = x.shape
    e = edge_index.shape[1]
    dh = d // NC
    row = edge_index[0]
    col = edge_index[1]

    ept = ((e + NS * CHUNK - 1) // (NS * CHUNK)) * CHUNK
    nchunks = ept // CHUNK
    nchunks = ((nchunks + 15) // 16) * 16
    ept = nchunks * CHUNK
    e_pad = ept * NS
    nch_sc = nchunks // 2
    npt = ((n + NS * L - 1) // (NS * L)) * L
    n_pad = npt * NS

    row_p = jnp.pad(row, (0, e_pad - e)).reshape(NS, nchunks, CHUNK)
    col_p = jnp.pad(col, (0, e_pad - e)).reshape(NS, nchunks, CHUNK)
    ea_p = jnp.pad(edge_attr, ((0, e_pad - e), (0, 0)))
    x_split = jnp.pad(
        jnp.stack([x[:, i * dh:(i + 1) * dh] for i in range(NC)], axis=0),
        ((0, 0), (0, n_pad - n), (0, 0)))

    eas = pl.pallas_call(
        _eas_body,
        out_shape=jax.ShapeDtypeStruct((e_pad // 32, 32), jnp.float32),
    )(ea_p.reshape(e_pad // 32, 128))
    eas_w = eas.reshape(NS, nchunks, CHUNK)

    deg_part = pl.kernel(
        functools.partial(_deg_kernel, n_pad, npt, nch_sc),
        out_type=jax.ShapeDtypeStruct((NC, n_pad), jnp.float32),
        mesh=plsc.VectorSubcoreMesh(**_MESH),
        compiler_params=pltpu.CompilerParams(needs_layout_passes=False, use_tc_tiling_on_sc=False),
        scratch_types=[
            pltpu.VMEM((nch_sc, CHUNK), jnp.int32),
            pltpu.VMEM((nch_sc, CHUNK), jnp.float32),
            pltpu.VMEM((npt,), jnp.float32),
            pltpu.VMEM_SHARED((n_pad,), jnp.float32),
        ],
    )(col_p, eas_w)

    dis = pl.pallas_call(
        _dis_body,
        out_shape=jax.ShapeDtypeStruct((n_pad // 128, 128), jnp.float32),
    )(deg_part.reshape(2 * (n_pad // 128), 128)).reshape(n_pad)

    w = pl.kernel(
        functools.partial(_w_kernel, nch_sc),
        out_type=jax.ShapeDtypeStruct((NS, nchunks, CHUNK), jnp.float32),
        mesh=plsc.VectorSubcoreMesh(**_MESH),
        compiler_params=pltpu.CompilerParams(needs_layout_passes=False, use_tc_tiling_on_sc=False),
        scratch_types=[
            pltpu.VMEM((nch_sc, CHUNK), jnp.int32),
            pltpu.VMEM((nch_sc, CHUNK), jnp.int32),
            pltpu.VMEM((nch_sc, CHUNK), jnp.float32),
            pltpu.VMEM((n_pad,), jnp.float32),
            pltpu.VMEM((nch_sc, CHUNK), jnp.float32),
        ],
    )(row_p, col_p, eas_w, dis)

    NBUF = 4
    NSEG = 2
    nch_seg = nchunks // NSEG

    def _hop_body(x_hbm, row_hbm, col_hbm, w_hbm, part_hbm,
                  row_v, col_v, w_v, bufs, gsems, ssems, acc_sh):
        cid = lax.axis_index("c")
        sid = lax.axis_index("s")

        def zfill(i, _):
            for g in range(dh // L):
                bufs[0][i, pl.ds(g * L, L)] = jnp.zeros((L,), jnp.float32)
            return ()
        lax.fori_loop(0, CHUNK, zfill, (), unroll=False)

        def zbody(i, _):
            pltpu.sync_copy(
                bufs[0], acc_sh.at[pl.ds(sid * npt + i * CHUNK, CHUNK)])
            return ()
        lax.fori_loop(0, npt // CHUNK, zbody, (), unroll=False)
        plsc.subcore_barrier()

        xc = x_hbm.at[cid]

        def scale(j, buf):
            @plsc.parallel_loop(0, CHUNK, step=1, unroll=8)
            def _(r):
                wb = plsc.load_gather(w_v, [_full16(j), _full16(r)])
                for dd in range(dh // L):
                    sl = pl.ds(dd * L, L)
                    buf[r, sl] = buf[r, sl] * wb

        for seg in range(NSEG):
            cbase = seg * nch_seg
            pltpu.sync_copy(row_hbm.at[sid].at[pl.ds(cbase, nch_seg)], row_v)
            pltpu.sync_copy(col_hbm.at[sid].at[pl.ds(cbase, nch_seg)], col_v)
            pltpu.sync_copy(w_hbm.at[sid].at[pl.ds(cbase, nch_seg)], w_v)

            for b in range(NBUF - 1):
                pltpu.async_copy(xc.at[row_v.at[b]], bufs[b], gsems[b])

            def quad(jj, _):
                for b in range(NBUF):
                    j = jj * NBUF + b
                    buf, gs, ss = bufs[b], gsems[b], ssems[b]
                    pb = (b + NBUF - 1) % NBUF

                    def wait_prev():
                        pltpu.make_async_copy(
                            bufs[pb], acc_sh.at[col_v.at[j - 1]],
                            ssems[pb]).wait()

                    def prefetch():
                        pltpu.async_copy(
                            xc.at[row_v.at[j + NBUF - 1]], bufs[pb],
                            gsems[pb])

                    if b == 0:
                        @pl.when(jj >= 1)
                        def _():
                            wait_prev()
                        prefetch()
                    else:
                        wait_prev()

                        @pl.when(jj < nch_seg // NBUF - 1)
                        def _():
                            prefetch()

                    pltpu.make_async_copy(xc.at[row_v.at[j]], buf, gs).wait()
                    scale(j, buf)
                    pltpu.async_copy(buf, acc_sh.at[col_v.at[j]], ss,
                                     add=True)
                return ()
            lax.fori_loop(0, nch_seg // NBUF, quad, (), unroll=False)
            lb = (nch_seg - 1) % NBUF
            pltpu.make_async_copy(
                bufs[lb], acc_sh.at[col_v.at[nch_seg - 1]],
                ssems[lb]).wait()

        plsc.subcore_barrier()
        pltpu.sync_copy(acc_sh.at[pl.ds(sid * npt, npt)],
                        part_hbm.at[cid, pl.ds(sid * npt, npt)])

    hop = pl.kernel(
        _hop_body,
        out_type=jax.ShapeDtypeStruct((NC, n_pad, dh), jnp.float32),
        mesh=plsc.VectorSubcoreMesh(**_MESH),
        compiler_params=pltpu.CompilerParams(needs_layout_passes=False, use_tc_tiling_on_sc=False),
        scratch_types=[
            pltpu.VMEM((nch_seg, CHUNK), jnp.int32),
            pltpu.VMEM((nch_seg, CHUNK), jnp.int32),
            pltpu.VMEM((nch_seg, CHUNK), jnp.float32),
            [pltpu.VMEM((CHUNK, dh), jnp.float32) for _ in range(NBUF)],
            [pltpu.SemaphoreType.DMA for _ in range(NBUF)],
            [pltpu.SemaphoreType.DMA for _ in range(NBUF)],
            pltpu.VMEM_SHARED((n_pad, dh), jnp.float32),
        ],
    )

    xs = [x]
    cur = x_split
    for _ in range(K_HOPS):
        cur = hop(cur, row_p, col_p, w)
        xs.append(jnp.concatenate([cur[i, :n] for i in range(NC)], axis=1))
    return jnp.stack(xs, axis=0)
